# Initial kernel scaffold; baseline (speedup 1.0000x reference)
import functools

import jax
import jax.numpy as jnp
from jax import lax
from jax.experimental import pallas as pl
from jax.experimental.pallas import tpu as pltpu
from jax.experimental.pallas import tpu_sc as plsc

NC, NS, LANES = 2, 16, 16
NW = NC * NS
DIM = 4
CHUNK = 128
UNROLL = 10


@functools.lru_cache(maxsize=None)
def _build(B, L, V):
    rows_per_w = B // NW
    n_chunks = rows_per_w // CHUNK
    idx_n = CHUNK * L

    mesh = plsc.VectorSubcoreMesh(core_axis_name="c", subcore_axis_name="s")

    @functools.partial(
        pl.kernel,
        out_type=jax.ShapeDtypeStruct((B * DIM,), jnp.float32),
        mesh=mesh,
        scratch_types=[
            pltpu.VMEM((2, idx_n), jnp.int32),
            pltpu.VMEM((2, idx_n, DIM), jnp.float32),
            pltpu.VMEM((CHUNK * DIM,), jnp.float32),
            pltpu.SemaphoreType.DMA,
        ],
    )
    def kern(tags_hbm, table_hbm, out_hbm, idx_v, rows_v, out_v, sem):
        wid = lax.axis_index("s") * NC + lax.axis_index("c")
        tag_base = wid * rows_per_w * L
        out_base = wid * rows_per_w * DIM

        iota = lax.iota(jnp.int32, 16)
        colpat = iota % DIM
        rowpat50 = (iota // DIM) * L

        def start_gather(c):
            b = c % 2
            pltpu.sync_copy(
                tags_hbm.at[pl.ds(tag_base + c * idx_n, idx_n)], idx_v.at[b])
            return pltpu.async_copy(table_hbm.at[idx_v.at[b]], rows_v.at[b], sem)

        def compute(c):
            b = c % 2
            rows = rows_v.at[b]

            def q_body(q, _):
                ridx0 = rowpat50 + q * (4 * L)

                def l_body(i, carry):
                    acc0, acc1, ridx = carry
                    for j in range(UNROLL):
                        v = plsc.load_gather(rows, [ridx + j, colpat])
                        if j % 2 == 0:
                            acc0 = acc0 + v
                        else:
                            acc1 = acc1 + v
                    return acc0, acc1, ridx + UNROLL

                z = jnp.zeros((16,), jnp.float32)
                acc0, acc1, _ = lax.fori_loop(
                    0, L // UNROLL, l_body, (z, z, ridx0), unroll=False)
                out_v[pl.ds(q * 16, 16)] = acc0 + acc1
                return 0

            lax.fori_loop(0, CHUNK // 4, q_body, 0, unroll=False)
            pltpu.sync_copy(
                out_v, out_hbm.at[pl.ds(out_base + c * CHUNK * DIM, CHUNK * DIM)])

        pending = start_gather(0)
        for c in range(n_chunks):
            nxt = start_gather(c + 1) if c + 1 < n_chunks else None
            pending.wait()
            compute(c)
            pending = nxt

    return kern


def kernel(itemtags, table):
    B, L = itemtags.shape
    V, _ = table.shape
    kern = _build(B, L, V)
    out = kern(itemtags.reshape(B * L), table)
    return out.reshape(B, DIM)


# speedup vs baseline: 16.6511x; 16.6511x over previous
import functools

import jax
import jax.numpy as jnp
from jax import lax
from jax.experimental import pallas as pl
from jax.experimental.pallas import tpu as pltpu
from jax.experimental.pallas import tpu_sc as plsc

NC, NS, LANES = 2, 16, 16
NW = NC * NS
DIM = 4
CHUNK = 128
UNROLL = 10


@functools.lru_cache(maxsize=None)
def _build(B, L, V):
    rows_per_w = B // NW
    n_chunks = rows_per_w // CHUNK
    idx_n = CHUNK * L

    mesh = plsc.VectorSubcoreMesh(core_axis_name="c", subcore_axis_name="s")

    @functools.partial(
        pl.kernel,
        out_type=jax.ShapeDtypeStruct((B * DIM,), jnp.float32),
        mesh=mesh,
        scratch_types=[
            pltpu.VMEM((2, idx_n), jnp.int32),
            pltpu.VMEM((2, idx_n, DIM), jnp.float32),
            pltpu.VMEM((CHUNK * DIM,), jnp.float32),
            pltpu.SemaphoreType.DMA,
        ],
        compiler_params=pltpu.CompilerParams(
            needs_layout_passes=False, use_tc_tiling_on_sc=False),
    )
    def kern(tags_hbm, table_hbm, out_hbm, idx_v, rows_v, out_v, sem):
        wid = lax.axis_index("s") * NC + lax.axis_index("c")
        tag_base = wid * rows_per_w * L
        out_base = wid * rows_per_w * DIM

        iota = lax.iota(jnp.int32, 16)
        colpat = iota % DIM
        rowpat50 = (iota // DIM) * L

        def start_gather(c):
            b = c % 2
            pltpu.sync_copy(
                tags_hbm.at[pl.ds(tag_base + c * idx_n, idx_n)], idx_v.at[b])
            return pltpu.async_copy(table_hbm.at[idx_v.at[b]], rows_v.at[b], sem)

        def compute(c):
            b = c % 2
            rows = rows_v.at[b]

            def q_body(q, _):
                ridx0 = rowpat50 + q * (4 * L)

                def l_body(i, carry):
                    acc0, acc1, ridx = carry
                    for j in range(UNROLL):
                        v = plsc.load_gather(rows, [ridx + j, colpat])
                        if j % 2 == 0:
                            acc0 = acc0 + v
                        else:
                            acc1 = acc1 + v
                    return acc0, acc1, ridx + UNROLL

                z = jnp.zeros((16,), jnp.float32)
                acc0, acc1, _ = lax.fori_loop(
                    0, L // UNROLL, l_body, (z, z, ridx0), unroll=False)
                out_v[pl.ds(q * 16, 16)] = acc0 + acc1
                return 0

            lax.fori_loop(0, CHUNK // 4, q_body, 0, unroll=False)
            pltpu.sync_copy(
                out_v, out_hbm.at[pl.ds(out_base + c * CHUNK * DIM, CHUNK * DIM)])

        pending = start_gather(0)
        for c in range(n_chunks):
            nxt = start_gather(c + 1) if c + 1 < n_chunks else None
            pending.wait()
            compute(c)
            pending = nxt

    return kern


def kernel(itemtags, table):
    B, L = itemtags.shape
    V, _ = table.shape
    kern = _build(B, L, V)
    out = kern(itemtags.reshape(B * L), table)
    return out.reshape(B, DIM)
